# baseline (speedup 1.0000x reference)
import functools

import jax
import jax.numpy as jnp
from jax.experimental import pallas as pl
from jax.experimental.pallas import tpu as pltpu

_TOPK = 8
_CHUNK = 64


def _loss_kernel(tw_ref, s_ref, t_ref, g_ref, o1_ref, o2_ref, o3_ref,
                 a1_ref, a2_ref, gm_ref, *, nj):
    j = pl.program_id(0)
    R = s_ref.shape[1]
    B = s_ref.shape[2]
    niter = R // _CHUNK
    nsub = _CHUNK // 8

    def body(it, carry):
        a1, a2, gm = carry
        base = pl.multiple_of(it * _CHUNK, _CHUNK)
        s = s_ref[0, pl.ds(base, _CHUNK), :]
        t = t_ref[0, pl.ds(base, _CHUNK), :]
        g = g_ref[0, pl.ds(base, _CHUNK), :]
        d1 = s - g
        d2 = s - t
        a1 = a1 + jnp.sum((d1 * d1).reshape(nsub, 8, B), axis=0)
        a2 = a2 + jnp.sum((d2 * d2).reshape(nsub, 8, B), axis=0)
        gm = jnp.maximum(gm, jnp.max(g.reshape(nsub, 8, B), axis=0))
        return a1, a2, gm

    zero = jnp.zeros((8, B), jnp.float32)
    ninf = jnp.full((8, B), -jnp.inf, jnp.float32)
    a1, a2, gm = jax.lax.fori_loop(0, niter, body, (zero, zero, ninf))
    a1_ref[pl.ds(j, 1), :] = jnp.sum(a1, axis=0, keepdims=True)
    a2_ref[pl.ds(j, 1), :] = jnp.sum(a2, axis=0, keepdims=True)
    gm_ref[pl.ds(j, 1), :] = jnp.max(gm, axis=0, keepdims=True)

    @pl.when(j == nj - 1)
    def _epilogue():
        J, Bq = a1_ref.shape
        HW = R
        tw = tw_ref[...]
        tw2 = tw * tw
        A1 = a1_ref[...]
        A2 = a2_ref[...]
        gmax = jnp.max(gm_ref[...], axis=1, keepdims=True)
        notc = jnp.where(gmax == 1.0, 0.0, 1.0)
        wl = tw2 * (A1 + notc * A2)
        mse = jnp.sum(wl) / (Bq * HW)
        lm = (0.5 / HW) * wl
        iota = jax.lax.broadcasted_iota(jnp.int32, (J, Bq), 0)
        acc = jnp.zeros((1, Bq), jnp.float32)
        cur = lm
        for _ in range(_TOPK):
            m = jnp.max(cur, axis=0, keepdims=True)
            acc = acc + m
            first = jnp.min(jnp.where(cur == m, iota, J), axis=0,
                            keepdims=True)
            cur = jnp.where(iota == first, -jnp.inf, cur)
        ohkm = jnp.sum(acc) / (_TOPK * Bq)
        o1_ref[0, 0] = ohkm
        o2_ref[0, 0] = mse / J
        o3_ref[0, 0] = ohkm + mse


def kernel(output_s, output_t, target, target_weight):
    B, J, H, W = output_s.shape
    HW = H * W
    st = jnp.transpose(output_s, (1, 2, 3, 0)).reshape(J, HW, B)
    tt = jnp.transpose(output_t, (1, 2, 3, 0)).reshape(J, HW, B)
    gt = jnp.transpose(target, (1, 2, 3, 0)).reshape(J, HW, B)
    twt = jnp.transpose(target_weight.reshape(B, J))
    scalar = jax.ShapeDtypeStruct((1, 1), jnp.float32)
    smem_spec = pl.BlockSpec(memory_space=pltpu.SMEM)
    o1, o2, o3 = pl.pallas_call(
        functools.partial(_loss_kernel, nj=J),
        grid=(J,),
        in_specs=[
            pl.BlockSpec((J, B), lambda j: (0, 0)),
            pl.BlockSpec((1, HW, B), lambda j: (j, 0, 0)),
            pl.BlockSpec((1, HW, B), lambda j: (j, 0, 0)),
            pl.BlockSpec((1, HW, B), lambda j: (j, 0, 0)),
        ],
        out_specs=[smem_spec, smem_spec, smem_spec],
        out_shape=[scalar, scalar, scalar],
        scratch_shapes=[
            pltpu.VMEM((J, B), jnp.float32),
            pltpu.VMEM((J, B), jnp.float32),
            pltpu.VMEM((J, B), jnp.float32),
        ],
    )(twt, st, tt, gt)
    return (o1[0, 0], o2[0, 0], o3[0, 0])
